# Initial kernel scaffold; baseline (speedup 1.0000x reference)
import jax
import jax.numpy as jnp
from jax.experimental import pallas as pl


def kernel(x, table):
    raise NotImplementedError("write your pallas kernel here")


# speedup vs baseline: 3.1139x; 3.1139x over previous
import jax
import jax.numpy as jnp
from jax.experimental import pallas as pl
from jax.experimental.pallas import tpu as pltpu
from jax.experimental.pallas import tpu_sc as plsc


def kernel(x, table):
    b, l = x.shape
    _, d = table.shape
    n = b * l
    window = 128
    assert n % window == 0

    idx = x.reshape(1, n)
    mesh = plsc.VectorSubcoreMesh(core_axis_name="core", subcore_axis_name="subcore")

    @pl.kernel(out_type=jax.ShapeDtypeStruct((n, d), table.dtype), mesh=mesh)
    def gather_kernel(tab_hbm, i_hbm, o_hbm):
        def body(i_vmem, o_vmem):
            pltpu.sync_copy(tab_hbm.at[i_vmem.at[0]], o_vmem)

        pltpu.emit_pipeline(
            body,
            grid=(n // window,),
            in_specs=[pl.BlockSpec((1, window), index_map=lambda i: (0, i))],
            out_specs=[pl.BlockSpec((window, d), index_map=lambda i: (i, 0))],
            core_axis_name=("core", "subcore"),
            dimension_semantics=(pltpu.PARALLEL,),
        )(i_hbm, o_hbm)

    out = gather_kernel(table, idx)
    return out.reshape(b, l, d)


# speedup vs baseline: 3.2971x; 1.0588x over previous
import jax
import jax.numpy as jnp
from jax.experimental import pallas as pl
from jax.experimental.pallas import tpu as pltpu
from jax.experimental.pallas import tpu_sc as plsc


def kernel(x, table):
    b, l = x.shape
    _, d = table.shape
    n = b * l
    window = 256
    assert n % window == 0

    idx = x.reshape(1, n)
    mesh = plsc.VectorSubcoreMesh(core_axis_name="core", subcore_axis_name="subcore")

    @pl.kernel(out_type=jax.ShapeDtypeStruct((n, d), table.dtype), mesh=mesh)
    def gather_kernel(tab_hbm, i_hbm, o_hbm):
        def body(i_vmem, o_vmem):
            pltpu.sync_copy(tab_hbm.at[i_vmem.at[0]], o_vmem)

        pltpu.emit_pipeline(
            body,
            grid=(n // window,),
            in_specs=[pl.BlockSpec((1, window), index_map=lambda i: (0, i))],
            out_specs=[pl.BlockSpec((window, d), index_map=lambda i: (i, 0))],
            core_axis_name=("core", "subcore"),
            dimension_semantics=(pltpu.PARALLEL,),
        )(i_hbm, o_hbm)

    out = gather_kernel(table, idx)
    return out.reshape(b, l, d)


# speedup vs baseline: 4.2401x; 1.2860x over previous
import jax
import jax.numpy as jnp
from jax.experimental import pallas as pl
from jax.experimental.pallas import tpu as pltpu
from jax.experimental.pallas import tpu_sc as plsc


def kernel(x, table):
    b, l = x.shape
    _, d = table.shape
    rows = 8
    assert b % rows == 0

    mesh = plsc.VectorSubcoreMesh(core_axis_name="core", subcore_axis_name="subcore")

    @pl.kernel(out_type=jax.ShapeDtypeStruct((b, l, d), table.dtype), mesh=mesh)
    def gather_kernel(tab_hbm, i_hbm, o_hbm):
        def body(i_vmem, o_vmem):
            @pl.loop(0, rows)
            def _(r):
                pltpu.sync_copy(tab_hbm.at[i_vmem.at[r]], o_vmem.at[r])

        pltpu.emit_pipeline(
            body,
            grid=(b // rows,),
            in_specs=[pl.BlockSpec((rows, l), index_map=lambda i: (i, 0))],
            out_specs=[pl.BlockSpec((rows, l, d), index_map=lambda i: (i, 0, 0))],
            core_axis_name=("core", "subcore"),
            dimension_semantics=(pltpu.PARALLEL,),
        )(i_hbm, o_hbm)

    return gather_kernel(table, x)


# speedup vs baseline: 5.8985x; 1.3911x over previous
import functools

import jax
from jax import lax
import jax.numpy as jnp
from jax.experimental import pallas as pl
from jax.experimental.pallas import tpu as pltpu
from jax.experimental.pallas import tpu_sc as plsc

_NUM_CORES = 2
_NUM_SUBCORES = 16


def kernel(x, table):
    b, l = x.shape
    _, d = table.shape
    nw = _NUM_CORES * _NUM_SUBCORES
    b_per_w = b // nw
    chunk = 8
    nchunks = b_per_w // chunk
    assert b_per_w * nw == b and chunk * nchunks == b_per_w and nchunks % 2 == 0

    mesh = plsc.VectorSubcoreMesh(core_axis_name="c", subcore_axis_name="s")

    @functools.partial(
        pl.kernel,
        mesh=mesh,
        out_type=jax.ShapeDtypeStruct((b, l, d), table.dtype),
        scratch_types=[
            pltpu.VMEM((b_per_w, l), jnp.int32),
            pltpu.VMEM((chunk, l, d), table.dtype),
            pltpu.VMEM((chunk, l, d), table.dtype),
            pltpu.SemaphoreType.DMA,
            pltpu.SemaphoreType.DMA,
            pltpu.SemaphoreType.DMA,
            pltpu.SemaphoreType.DMA,
        ],
    )
    def gather_kernel(tab_hbm, x_hbm, o_hbm, idx_v, buf0, buf1, g0, g1, o0, o1):
        wid = lax.axis_index("s") * _NUM_CORES + lax.axis_index("c")
        base = wid * b_per_w
        pltpu.sync_copy(x_hbm.at[pl.ds(base, b_per_w)], idx_v)

        bufs = (buf0, buf1)
        gsems = (g0, g1)
        osems = (o0, o1)

        @pl.loop(0, nchunks, step=2)
        def _(g):
            for bi in range(2):
                buf, gsem, osem = bufs[bi], gsems[bi], osems[bi]
                gg = g + bi

                @pl.when(gg >= 2)
                def _():
                    pltpu.make_async_copy(
                        buf, o_hbm.at[pl.ds(base, chunk)], osem
                    ).wait()

                copies = [
                    pltpu.async_copy(
                        tab_hbm.at[idx_v.at[gg * chunk + r]], buf.at[r], gsem
                    )
                    for r in range(chunk)
                ]
                for cp in copies:
                    cp.wait()
                pltpu.async_copy(
                    buf, o_hbm.at[pl.ds(base + gg * chunk, chunk)], osem
                )

        for bi in range(2):
            pltpu.make_async_copy(
                bufs[bi], o_hbm.at[pl.ds(base, chunk)], osems[bi]
            ).wait()

    return gather_kernel(table, x)
